# baseline (speedup 1.0000x reference)
import functools

import jax
import jax.numpy as jnp
from jax import lax
from jax.experimental import pallas as pl
from jax.experimental.pallas import tpu as pltpu
from jax.experimental.pallas import tpu_sc as plsc


def _bn(h, g, b):
    m = jnp.mean(h, axis=0, keepdims=True)
    v = jnp.mean((h - m) ** 2, axis=0, keepdims=True)
    return g * (h - m) / jnp.sqrt(v + 1e-5) + b


def _bn_in_body(x_ref, g_ref, b_ref, o_ref):
    o_ref[...] = _bn(x_ref[...], g_ref[...], b_ref[...])


def _gin_body(x0_ref, parts_ref, w1_ref, b1_ref, w2_ref, b2_ref, g_ref,
              bb_ref, emb_ref):
    x0 = x0_ref[...]
    o = parts_ref[0] + parts_ref[1] + x0
    h = jnp.dot(o, w1_ref[0], preferred_element_type=jnp.float32) + b1_ref[0]
    h = jnp.maximum(h, 0.0)
    h = jnp.dot(h, w2_ref[0], preferred_element_type=jnp.float32) + b2_ref[0]
    emb_ref[0] = jnp.tanh(_bn(h, g_ref[0], bb_ref[0]))


def _att_body(x_ref, emb_ref, wq_ref, bq_ref, wk_ref, bk_ref, wv_ref, bv_ref,
              w1_ref, b1_ref, g1_ref, be1_ref, w2_ref, b2_ref, g2_ref,
              be2_ref, w3_ref, b3_ref, out_ref):
    f32 = jnp.float32
    q = jnp.tanh(jnp.dot(x_ref[...], wq_ref[...], preferred_element_type=f32)
                 + bq_ref[...])
    scores = []
    for t in range(3):
        kt = jnp.tanh(jnp.dot(emb_ref[t], wk_ref[...],
                              preferred_element_type=f32) + bk_ref[...])
        scores.append(jnp.sum(kt * q, axis=1, keepdims=True))
    m = jnp.maximum(jnp.maximum(scores[0], scores[1]), scores[2])
    es = [jnp.exp(s - m) for s in scores]
    z = es[0] + es[1] + es[2]
    res = jnp.zeros_like(q)
    for t in range(3):
        vt = jnp.tanh(jnp.dot(emb_ref[t], wv_ref[...],
                              preferred_element_type=f32) + bv_ref[...])
        res = res + (es[t] / z) * vt
    h = jnp.dot(res, w1_ref[...], preferred_element_type=f32) + b1_ref[...]
    h = jnp.tanh(_bn(h, g1_ref[...], be1_ref[...]))
    h = jnp.dot(h, w2_ref[...], preferred_element_type=f32) + b2_ref[...]
    h = jnp.tanh(_bn(h, g2_ref[...], be2_ref[...]))
    out_ref[...] = jnp.dot(h, w3_ref[...], preferred_element_type=f32) \
        + b3_ref[...]


_NC = 2
_NS = 16
_NW = _NC * _NS
_C = 64
_G = 4
_Q = 4


def _make_segsum(n, d, k_chunks, n_acc):
    stripe = n_acc // _NS
    kq = k_chunks // _Q
    mesh = plsc.VectorSubcoreMesh(core_axis_name="c", subcore_axis_name="s")

    @functools.partial(
        pl.kernel,
        mesh=mesh,
        out_type=jax.ShapeDtypeStruct((6, n_acc, d), jnp.float32),
        scratch_types=[
            pltpu.VMEM((kq, _C), jnp.int32),
            pltpu.VMEM((kq, _C), jnp.int32),
            pltpu.VMEM((_G, _C, d), jnp.float32),
            pltpu.VMEM_SHARED((n_acc, d), jnp.float32),
            pltpu.SemaphoreType.DMA((_G,)),
        ],
    )
    def segsum(x0_hbm, zeros_hbm, src_hbm, dst_hbm, out_hbm,
               src_v, dst_v, rows_v, acc, gsem):
        c = lax.axis_index("c")
        s = lax.axis_index("s")
        wid = s * _NC + c
        row0 = s * stripe
        for t in range(3):
            pltpu.sync_copy(zeros_hbm.at[pl.ds(row0, stripe)],
                            acc.at[pl.ds(row0, stripe)])
            plsc.subcore_barrier()
            for q in range(_Q):
                pltpu.sync_copy(src_hbm.at[t, wid, pl.ds(q * kq, kq)],
                                src_v)
                pltpu.sync_copy(dst_hbm.at[t, wid, pl.ds(q * kq, kq)],
                                dst_v)

                def body(g, carry):
                    base = _G * g
                    for j in range(_G):
                        pltpu.async_copy(x0_hbm.at[src_v.at[base + j]],
                                         rows_v.at[j], gsem.at[j])
                    for j in range(_G):
                        pltpu.make_async_copy(
                            x0_hbm.at[src_v.at[base + j]],
                            rows_v.at[j], gsem.at[j]).wait()
                        pltpu.sync_copy(rows_v.at[j],
                                        acc.at[dst_v.at[base + j]],
                                        add=True)
                    return carry

                lax.fori_loop(0, kq // _G, body, 0)
            plsc.subcore_barrier()
            pltpu.sync_copy(acc.at[pl.ds(row0, stripe)],
                            out_hbm.at[2 * t + c, pl.ds(row0, stripe)])

    return segsum


def kernel(x, edge_index_p, edge_index_s, edge_index_v, params):
    n, d = x.shape
    e = edge_index_p.shape[1]
    h1 = params['c_W1'].shape[1]
    h2 = params['c_W2'].shape[1]

    k_chunks = -(-e // (_NW * _C))
    align = _Q * _G * 2
    k_chunks = ((k_chunks + align - 1) // align) * align
    e_pad = _NW * _C * k_chunks
    stripe = ((-(-n // _NS) + 7) // 8) * 8
    n_acc = stripe * _NS
    f32 = jnp.float32

    x0 = pl.pallas_call(
        _bn_in_body,
        out_shape=jax.ShapeDtypeStruct((n, d), f32),
    )(x, params['in_g'].reshape(1, d), params['in_b'].reshape(1, d))

    def _prep(ei):
        src = ei[0].astype(jnp.int32)
        dst = ei[1].astype(jnp.int32)
        pad = e_pad - e
        src = jnp.concatenate([src, jnp.zeros((pad,), jnp.int32)])
        dst = jnp.concatenate([dst, jnp.full((pad,), n, jnp.int32)])
        return src.reshape(_NW, k_chunks, _C), dst.reshape(_NW, k_chunks, _C)

    sp, dp = _prep(edge_index_p)
    ss, ds_ = _prep(edge_index_s)
    sv, dv = _prep(edge_index_v)
    srcs = jnp.stack([sp, ss, sv])
    dsts = jnp.stack([dp, ds_, dv])
    zeros = jnp.zeros((n_acc, d), f32)

    parts = _make_segsum(n, d, k_chunks, n_acc)(x0, zeros, srcs, dsts)

    gp = params['gin_p']; gs = params['gin_s']; gv = params['gin_v']
    w1s = jnp.stack([gp['W1'], gs['W1'], gv['W1']])
    b1s = jnp.stack([gp['b1'], gs['b1'], gv['b1']]).reshape(3, 1, d)
    w2s = jnp.stack([gp['W2'], gs['W2'], gv['W2']])
    b2s = jnp.stack([gp['b2'], gs['b2'], gv['b2']]).reshape(3, 1, d)
    gns = jnp.stack([params['p_g'], params['s_g'], params['v_g']]).reshape(3, 1, d)
    bns = jnp.stack([params['p_b'], params['s_b'], params['v_b']]).reshape(3, 1, d)

    emb = pl.pallas_call(
        _gin_body,
        grid=(3,),
        in_specs=[
            pl.BlockSpec((n, d), lambda t: (0, 0)),
            pl.BlockSpec((2, n, d), lambda t: (t, 0, 0)),
            pl.BlockSpec((1, d, d), lambda t: (t, 0, 0)),
            pl.BlockSpec((1, 1, d), lambda t: (t, 0, 0)),
            pl.BlockSpec((1, d, d), lambda t: (t, 0, 0)),
            pl.BlockSpec((1, 1, d), lambda t: (t, 0, 0)),
            pl.BlockSpec((1, 1, d), lambda t: (t, 0, 0)),
            pl.BlockSpec((1, 1, d), lambda t: (t, 0, 0)),
        ],
        out_specs=pl.BlockSpec((1, n, d), lambda t: (t, 0, 0)),
        out_shape=jax.ShapeDtypeStruct((3, n, d), f32),
    )(x0, parts, w1s, b1s, w2s, b2s, gns, bns)

    out = pl.pallas_call(
        _att_body,
        out_shape=jax.ShapeDtypeStruct((n, 2), f32),
    )(x, emb,
      params['Wq'], params['bq'].reshape(1, d),
      params['Wk'], params['bk'].reshape(1, d),
      params['Wv'], params['bv'].reshape(1, d),
      params['c_W1'], params['c_b1'].reshape(1, h1),
      params['c_g1'].reshape(1, h1), params['c_be1'].reshape(1, h1),
      params['c_W2'], params['c_b2'].reshape(1, h2),
      params['c_g2'].reshape(1, h2), params['c_be2'].reshape(1, h2),
      params['c_W3'], params['c_b3'].reshape(1, 2))
    return out


# speedup vs baseline: 1.4082x; 1.4082x over previous
import functools

import jax
import jax.numpy as jnp
from jax import lax
from jax.experimental import pallas as pl
from jax.experimental.pallas import tpu as pltpu
from jax.experimental.pallas import tpu_sc as plsc


def _bn(h, g, b):
    m = jnp.mean(h, axis=0, keepdims=True)
    v = jnp.mean((h - m) ** 2, axis=0, keepdims=True)
    return g * (h - m) / jnp.sqrt(v + 1e-5) + b


def _bn_in_body(x_ref, g_ref, b_ref, o_ref):
    o_ref[...] = _bn(x_ref[...], g_ref[...], b_ref[...])


def _gin_body(x0_ref, parts_ref, w1_ref, b1_ref, w2_ref, b2_ref, g_ref,
              bb_ref, emb_ref):
    x0 = x0_ref[...]
    o = parts_ref[0] + parts_ref[1] + x0
    h = jnp.dot(o, w1_ref[0], preferred_element_type=jnp.float32) + b1_ref[0]
    h = jnp.maximum(h, 0.0)
    h = jnp.dot(h, w2_ref[0], preferred_element_type=jnp.float32) + b2_ref[0]
    emb_ref[0] = jnp.tanh(_bn(h, g_ref[0], bb_ref[0]))


def _att_body(x_ref, emb_ref, wq_ref, bq_ref, wk_ref, bk_ref, wv_ref, bv_ref,
              w1_ref, b1_ref, g1_ref, be1_ref, w2_ref, b2_ref, g2_ref,
              be2_ref, w3_ref, b3_ref, out_ref):
    f32 = jnp.float32
    q = jnp.tanh(jnp.dot(x_ref[...], wq_ref[...], preferred_element_type=f32)
                 + bq_ref[...])
    scores = []
    for t in range(3):
        kt = jnp.tanh(jnp.dot(emb_ref[t], wk_ref[...],
                              preferred_element_type=f32) + bk_ref[...])
        scores.append(jnp.sum(kt * q, axis=1, keepdims=True))
    m = jnp.maximum(jnp.maximum(scores[0], scores[1]), scores[2])
    es = [jnp.exp(s - m) for s in scores]
    z = es[0] + es[1] + es[2]
    res = jnp.zeros_like(q)
    for t in range(3):
        vt = jnp.tanh(jnp.dot(emb_ref[t], wv_ref[...],
                              preferred_element_type=f32) + bv_ref[...])
        res = res + (es[t] / z) * vt
    h = jnp.dot(res, w1_ref[...], preferred_element_type=f32) + b1_ref[...]
    h = jnp.tanh(_bn(h, g1_ref[...], be1_ref[...]))
    h = jnp.dot(h, w2_ref[...], preferred_element_type=f32) + b2_ref[...]
    h = jnp.tanh(_bn(h, g2_ref[...], be2_ref[...]))
    out_ref[...] = jnp.dot(h, w3_ref[...], preferred_element_type=f32) \
        + b3_ref[...]


_NC = 2
_NS = 16
_NW = _NC * _NS
_C = 128


def _make_segsum(n, d, k_chunks, n_acc):
    stripe = n_acc // _NS
    mesh = plsc.VectorSubcoreMesh(core_axis_name="c", subcore_axis_name="s")

    @functools.partial(
        pl.kernel,
        mesh=mesh,
        out_type=jax.ShapeDtypeStruct((6, n_acc, d), jnp.float32),
        scratch_types=[
            pltpu.VMEM((k_chunks, _C), jnp.int32),
            pltpu.VMEM((k_chunks, _C), jnp.int32),
            pltpu.VMEM((_C, d), jnp.float32),
            pltpu.VMEM_SHARED((n_acc, d), jnp.float32),
            pltpu.SemaphoreType.DMA,
        ],
    )
    def segsum(x0_hbm, zeros_hbm, src_hbm, dst_hbm, out_hbm,
               src_v, dst_v, rows_v, acc, sem):
        c = lax.axis_index("c")
        s = lax.axis_index("s")
        wid = s * _NC + c
        row0 = s * stripe
        for t in range(3):
            pltpu.sync_copy(zeros_hbm.at[pl.ds(row0, stripe)],
                            acc.at[pl.ds(row0, stripe)])
            pltpu.sync_copy(src_hbm.at[t, wid], src_v)
            pltpu.sync_copy(dst_hbm.at[t, wid], dst_v)
            plsc.subcore_barrier()

            def body(k, carry):
                pltpu.async_copy(x0_hbm.at[src_v.at[k]], rows_v, sem).wait()
                pltpu.sync_copy(rows_v, acc.at[dst_v.at[k]], add=True)
                return carry

            lax.fori_loop(0, k_chunks, body, 0)
            plsc.subcore_barrier()
            pltpu.sync_copy(acc.at[pl.ds(row0, stripe)],
                            out_hbm.at[2 * t + c, pl.ds(row0, stripe)])

    return segsum


def kernel(x, edge_index_p, edge_index_s, edge_index_v, params):
    n, d = x.shape
    e = edge_index_p.shape[1]
    h1 = params['c_W1'].shape[1]
    h2 = params['c_W2'].shape[1]

    k_chunks = -(-e // (_NW * _C))
    e_pad = _NW * _C * k_chunks
    stripe = ((-(-n // _NS) + 7) // 8) * 8
    n_acc = stripe * _NS
    f32 = jnp.float32

    x0 = pl.pallas_call(
        _bn_in_body,
        out_shape=jax.ShapeDtypeStruct((n, d), f32),
    )(x, params['in_g'].reshape(1, d), params['in_b'].reshape(1, d))

    def _prep(ei):
        src = ei[0].astype(jnp.int32)
        dst = ei[1].astype(jnp.int32)
        pad = e_pad - e
        src = jnp.concatenate([src, jnp.zeros((pad,), jnp.int32)])
        dst = jnp.concatenate([dst, jnp.full((pad,), n, jnp.int32)])
        return src.reshape(_NW, k_chunks, _C), dst.reshape(_NW, k_chunks, _C)

    sp, dp = _prep(edge_index_p)
    ss, ds_ = _prep(edge_index_s)
    sv, dv = _prep(edge_index_v)
    srcs = jnp.stack([sp, ss, sv])
    dsts = jnp.stack([dp, ds_, dv])
    zeros = jnp.zeros((n_acc, d), f32)

    parts = _make_segsum(n, d, k_chunks, n_acc)(x0, zeros, srcs, dsts)

    gp = params['gin_p']; gs = params['gin_s']; gv = params['gin_v']
    w1s = jnp.stack([gp['W1'], gs['W1'], gv['W1']])
    b1s = jnp.stack([gp['b1'], gs['b1'], gv['b1']]).reshape(3, 1, d)
    w2s = jnp.stack([gp['W2'], gs['W2'], gv['W2']])
    b2s = jnp.stack([gp['b2'], gs['b2'], gv['b2']]).reshape(3, 1, d)
    gns = jnp.stack([params['p_g'], params['s_g'], params['v_g']]).reshape(3, 1, d)
    bns = jnp.stack([params['p_b'], params['s_b'], params['v_b']]).reshape(3, 1, d)

    emb = pl.pallas_call(
        _gin_body,
        grid=(3,),
        in_specs=[
            pl.BlockSpec((n, d), lambda t: (0, 0)),
            pl.BlockSpec((2, n, d), lambda t: (t, 0, 0)),
            pl.BlockSpec((1, d, d), lambda t: (t, 0, 0)),
            pl.BlockSpec((1, 1, d), lambda t: (t, 0, 0)),
            pl.BlockSpec((1, d, d), lambda t: (t, 0, 0)),
            pl.BlockSpec((1, 1, d), lambda t: (t, 0, 0)),
            pl.BlockSpec((1, 1, d), lambda t: (t, 0, 0)),
            pl.BlockSpec((1, 1, d), lambda t: (t, 0, 0)),
        ],
        out_specs=pl.BlockSpec((1, n, d), lambda t: (t, 0, 0)),
        out_shape=jax.ShapeDtypeStruct((3, n, d), f32),
    )(x0, parts, w1s, b1s, w2s, b2s, gns, bns)

    out = pl.pallas_call(
        _att_body,
        out_shape=jax.ShapeDtypeStruct((n, 2), f32),
    )(x, emb,
      params['Wq'], params['bq'].reshape(1, d),
      params['Wk'], params['bk'].reshape(1, d),
      params['Wv'], params['bv'].reshape(1, d),
      params['c_W1'], params['c_b1'].reshape(1, h1),
      params['c_g1'].reshape(1, h1), params['c_be1'].reshape(1, h1),
      params['c_W2'], params['c_b2'].reshape(1, h2),
      params['c_g2'].reshape(1, h2), params['c_be2'].reshape(1, h2),
      params['c_W3'], params['c_b3'].reshape(1, 2))
    return out
